# baseline (speedup 1.0000x reference)
import functools

import jax
import jax.numpy as jnp
from jax.experimental import pallas as pl
from jax.experimental.pallas import tpu as pltpu

NHI = 32
NLO = 64
NBINS = NHI * NLO
_LOG2_NLO = 6


def _pick_chunk(P):
    for ch in (9216, 4608, 2304, 1152, 768, 512, 384, 256, 128):
        if P % ch == 0:
            return ch
    return P


def _fused_kernel(x_ref, t_ref, out_ref, hist_ref, mx_ref, *, C, CH,
                  nstep_inner, nsteps):
    ph = pl.program_id(0)
    b = pl.program_id(1)
    i = pl.program_id(2)
    step = b * nstep_inner + i

    @pl.when((ph == 0) & (step == 0))
    def _():
        mx_ref[0, 0] = jnp.float32(0.0)
        hist_ref[...] = jnp.zeros_like(hist_ref)

    @pl.when(ph == 0)
    def _():
        m = jnp.max(jnp.abs(x_ref[0]))
        mx_ref[0, 0] = jnp.maximum(mx_ref[0, 0], m)

    @pl.when(ph == 1)
    def _():
        x = x_ref[0]
        t = t_ref[0]
        inv = 1.0 / (mx_ref[0, 0] + 1.0)

        iota_c = jax.lax.broadcasted_iota(jnp.int32, (C, CH), 0)
        mxv = jnp.max(t, axis=0, keepdims=True)
        lab = jnp.min(jnp.where(t == mxv, iota_c, C), axis=0, keepdims=True)

        fg_all = (lab == iota_c)
        e_all = jnp.abs(fg_all.astype(jnp.float32) - x)
        idx = (NBINS * (1.0 - e_all * inv)).astype(jnp.int32)
        idx = jnp.clip(idx, 0, NBINS - 1)
        hi_all = jnp.right_shift(idx, _LOG2_NLO)
        hi_all = jnp.where(fg_all, hi_all + NHI, hi_all)
        lo_all = jnp.bitwise_and(idx, NLO - 1)

        iota_hi = jax.lax.broadcasted_iota(jnp.int32, (2 * NHI, CH), 0)
        iota_lo = jax.lax.broadcasted_iota(jnp.int32, (NLO, CH), 0)
        for c in range(C):
            ohi = (hi_all[c:c + 1] == iota_hi).astype(jnp.bfloat16)
            olo = (lo_all[c:c + 1] == iota_lo).astype(jnp.bfloat16)
            upd = jax.lax.dot_general(
                ohi, olo, (((1,), (1,)), ((), ())),
                preferred_element_type=jnp.float32)
            hist_ref[c] = hist_ref[c] + upd

    @pl.when((ph == 1) & (step == nsteps - 1))
    def _():
        iu = jax.lax.broadcasted_iota(jnp.int32, (NLO, NLO), 0)
        ju = jax.lax.broadcasted_iota(jnp.int32, (NLO, NLO), 1)
        U = (iu <= ju).astype(jnp.float32)
        O = jnp.ones((NLO, NLO), jnp.float32)
        il = jax.lax.broadcasted_iota(jnp.int32, (NHI, NHI), 0)
        jl = jax.lax.broadcasted_iota(jnp.int32, (NHI, NHI), 1)
        L = (jl < il).astype(jnp.float32)

        def dn(a, bm):
            return jax.lax.dot_general(a, bm, (((1,), (0,)), ((), ())),
                                       preferred_element_type=jnp.float32)

        bidx = (jax.lax.broadcasted_iota(jnp.int32, (NHI, NLO), 0) * NLO +
                jax.lax.broadcasted_iota(jnp.int32, (NHI, NLO), 1))
        centers = ((1.0 - (bidx.astype(jnp.float32) + 0.5) / NBINS) *
                   (mx_ref[0, 0] + 1.0))

        total = jnp.float32(0.0)
        for c in range(C):
            h = hist_ref[c]
            g = h[NHI:2 * NHI]
            n = h[0:NHI] + g
            R = dn(n, U) + dn(L, dn(n, O))
            G = dn(g, U) + dn(L, dn(g, O))
            gts = jnp.sum(g)
            un_in = jnp.maximum(gts + R - G, 1.0)
            j_in = jnp.where(R > 0.5, 1.0 - (gts - G) / un_in, 0.0)
            Rx = R - n
            Gx = G - g
            un_ex = jnp.maximum(gts + Rx - Gx, 1.0)
            j_ex = jnp.where(Rx > 0.5, 1.0 - (gts - Gx) / un_ex, 0.0)
            total = total + jnp.sum(centers * (j_in - j_ex))
        out_ref[...] = jnp.broadcast_to(total / C, (1, 1))


def _lovasz_pallas(x, t, interpret=False):
    Bq, C, P = x.shape
    CH = _pick_chunk(P)
    nstep_inner = P // CH
    nsteps = Bq * nstep_inner
    out = pl.pallas_call(
        functools.partial(_fused_kernel, C=C, CH=CH,
                          nstep_inner=nstep_inner, nsteps=nsteps),
        grid=(2, Bq, nstep_inner),
        in_specs=[
            pl.BlockSpec((1, C, CH), lambda ph, b, i: (b, 0, i)),
            pl.BlockSpec((1, C, CH), lambda ph, b, i: (b * ph, 0, i * ph)),
        ],
        out_specs=pl.BlockSpec((1, 1), lambda ph, b, i: (0, 0)),
        out_shape=jax.ShapeDtypeStruct((1, 1), jnp.float32),
        scratch_shapes=[
            pltpu.VMEM((C, 2 * NHI, NLO), jnp.float32),
            pltpu.SMEM((1, 1), jnp.float32),
        ],
        interpret=interpret,
    )(x, t)
    return out[0, 0]


def kernel(pred, score, target):
    del score
    Bq, C = pred.shape[1], pred.shape[2]
    P = pred.shape[3] * pred.shape[4]
    x = pred.reshape(Bq, C, P).astype(jnp.float32)
    t = target.reshape(Bq, C, P).astype(jnp.float32)
    return _lovasz_pallas(x, t)


# speedup vs baseline: 1.0535x; 1.0535x over previous
import functools

import jax
import jax.numpy as jnp
from jax.experimental import pallas as pl
from jax.experimental.pallas import tpu as pltpu

NHI = 32
NLO = 64
NBINS = NHI * NLO
_LOG2_NLO = 6


def _pick_chunk(P):
    for ch in (18432, 9216, 4608, 2304, 1152, 768, 512, 384, 256, 128):
        if P % ch == 0:
            return ch
    return P


def _hist_kernel(inv_ref, x_ref, t_ref, hist_ref, *, C, CH):
    i = pl.program_id(1)

    @pl.when(i == 0)
    def _():
        hist_ref[...] = jnp.zeros_like(hist_ref)

    x = x_ref[0]
    t = t_ref[0]
    inv = inv_ref[0, 0]

    iota_c = jax.lax.broadcasted_iota(jnp.int32, (C, CH), 0)
    mxv = jnp.max(t, axis=0, keepdims=True)
    lab = jnp.min(jnp.where(t == mxv, iota_c, C), axis=0, keepdims=True)

    fg_all = (lab == iota_c)
    e_all = jnp.abs(fg_all.astype(jnp.float32) - x)
    idx = (NBINS * (1.0 - e_all * inv)).astype(jnp.int32)
    idx = jnp.clip(idx, 0, NBINS - 1)
    hi_all = jnp.right_shift(idx, _LOG2_NLO)
    hi_all = jnp.where(fg_all, hi_all + NHI, hi_all)
    lo_all = jnp.bitwise_and(idx, NLO - 1)

    iota_hi = jax.lax.broadcasted_iota(jnp.int32, (2 * NHI, CH), 0)
    iota_lo = jax.lax.broadcasted_iota(jnp.int32, (NLO, CH), 0)
    for c in range(C):
        ohi = (hi_all[c:c + 1] == iota_hi).astype(jnp.bfloat16)
        olo = (lo_all[c:c + 1] == iota_lo).astype(jnp.bfloat16)
        upd = jax.lax.dot_general(
            ohi, olo, (((1,), (1,)), ((), ())),
            preferred_element_type=jnp.float32)
        hist_ref[0, c] = hist_ref[0, c] + upd


def _reduce_kernel(inv_ref, hist_ref, out_ref, *, C, NB):
    iu = jax.lax.broadcasted_iota(jnp.int32, (NLO, NLO), 0)
    ju = jax.lax.broadcasted_iota(jnp.int32, (NLO, NLO), 1)
    U = (iu <= ju).astype(jnp.float32)
    O = jnp.ones((NLO, NLO), jnp.float32)
    il = jax.lax.broadcasted_iota(jnp.int32, (NHI, NHI), 0)
    jl = jax.lax.broadcasted_iota(jnp.int32, (NHI, NHI), 1)
    L = (jl < il).astype(jnp.float32)

    def dn(a, bm):
        return jax.lax.dot_general(a, bm, (((1,), (0,)), ((), ())),
                                   preferred_element_type=jnp.float32)

    bidx = (jax.lax.broadcasted_iota(jnp.int32, (NHI, NLO), 0) * NLO +
            jax.lax.broadcasted_iota(jnp.int32, (NHI, NLO), 1))
    centers = ((1.0 - (bidx.astype(jnp.float32) + 0.5) / NBINS) /
               inv_ref[0, 0])

    total = jnp.float32(0.0)
    for c in range(C):
        h = hist_ref[0, c]
        for b in range(1, NB):
            h = h + hist_ref[b, c]
        g = h[NHI:2 * NHI]
        n = h[0:NHI] + g
        R = dn(n, U) + dn(L, dn(n, O))
        G = dn(g, U) + dn(L, dn(g, O))
        gts = jnp.sum(g)
        un_in = jnp.maximum(gts + R - G, 1.0)
        j_in = jnp.where(R > 0.5, 1.0 - (gts - G) / un_in, 0.0)
        Rx = R - n
        Gx = G - g
        un_ex = jnp.maximum(gts + Rx - Gx, 1.0)
        j_ex = jnp.where(Rx > 0.5, 1.0 - (gts - Gx) / un_ex, 0.0)
        total = total + jnp.sum(centers * (j_in - j_ex))
    out_ref[...] = jnp.broadcast_to(total / C, (1, 1))


def _lovasz_pallas(x, t, interpret=False):
    Bq, C, P = x.shape
    CH = _pick_chunk(P)
    nstep_inner = P // CH
    inv = (1.0 / (jnp.max(jnp.abs(x)) + 1.0)).reshape(1, 1)
    hist = pl.pallas_call(
        functools.partial(_hist_kernel, C=C, CH=CH),
        grid=(Bq, nstep_inner),
        in_specs=[
            pl.BlockSpec(memory_space=pltpu.SMEM),
            pl.BlockSpec((1, C, CH), lambda b, i: (b, 0, i)),
            pl.BlockSpec((1, C, CH), lambda b, i: (b, 0, i)),
        ],
        out_specs=pl.BlockSpec((1, C, 2 * NHI, NLO), lambda b, i: (b, 0, 0, 0)),
        out_shape=jax.ShapeDtypeStruct((Bq, C, 2 * NHI, NLO), jnp.float32),
        interpret=interpret,
    )(inv, x, t)
    out = pl.pallas_call(
        functools.partial(_reduce_kernel, C=C, NB=Bq),
        in_specs=[
            pl.BlockSpec(memory_space=pltpu.SMEM),
            pl.BlockSpec(memory_space=pltpu.VMEM),
        ],
        out_shape=jax.ShapeDtypeStruct((1, 1), jnp.float32),
        interpret=interpret,
    )(inv, hist)
    return out[0, 0]


def kernel(pred, score, target):
    del score
    Bq, C = pred.shape[1], pred.shape[2]
    P = pred.shape[3] * pred.shape[4]
    x = pred.reshape(Bq, C, P).astype(jnp.float32)
    t = target.reshape(Bq, C, P).astype(jnp.float32)
    return _lovasz_pallas(x, t)


# speedup vs baseline: 1.1972x; 1.1364x over previous
import functools

import jax
import jax.numpy as jnp
from jax import lax
from jax.experimental import pallas as pl
from jax.experimental.pallas import tpu as pltpu
from jax.experimental.pallas import tpu_sc as plsc

NHI = 32
NLO = 64
NBINS = NHI * NLO
_LOG2_NLO = 6

_NC = 2
_NS = 16
_NW = _NC * _NS


def _pick_chunk(P):
    for ch in (18432, 9216, 4608, 2304, 1152, 768, 512, 384, 256, 128):
        if P % ch == 0:
            return ch
    return P


def _bin_kernel(inv_ref, x_ref, t_ref, gidx_ref, *, C, CH):
    x = x_ref[0]
    t = t_ref[0]
    inv = inv_ref[0, 0]

    iota_c = jax.lax.broadcasted_iota(jnp.int32, (C, CH), 0)
    mxv = jnp.max(t, axis=0, keepdims=True)
    lab = jnp.min(jnp.where(t == mxv, iota_c, C), axis=0, keepdims=True)

    fg_all = (lab == iota_c)
    e_all = jnp.abs(fg_all.astype(jnp.float32) - x)
    idx = (NBINS * (1.0 - e_all * inv)).astype(jnp.int32)
    idx = jnp.clip(idx, 0, NBINS - 1)
    idx = jnp.where(fg_all, idx + NBINS, idx)
    gidx_ref[0] = idx + iota_c * (2 * NBINS)


def _sc_hist_kernel(gidx_hbm, out_hbm, idx_buf, ones_buf, tmp_buf, shist,
                    *, hbins, items_w, chunk, nchunk):
    cid = lax.axis_index("c")
    sid = lax.axis_index("s")
    wid = sid * _NC + cid
    base = wid * items_w
    zeros16 = jnp.zeros((16,), jnp.float32)
    ones16 = jnp.full((16,), 1.0, jnp.float32)

    def zbody(i, carry):
        tmp_buf[pl.ds(i * 16, 16)] = zeros16
        return carry

    lax.fori_loop(0, hbins // 16, zbody, 0)

    def obody(i, carry):
        ones_buf[pl.ds(i * 16, 16)] = ones16
        return carry

    lax.fori_loop(0, chunk // 16, obody, 0)

    @pl.when(sid == 0)
    def _():
        pltpu.sync_copy(tmp_buf, shist)

    plsc.subcore_barrier()

    def chunk_body(j, carry):
        pltpu.sync_copy(gidx_hbm.at[pl.ds(base + j * chunk, chunk)],
                        idx_buf)
        pltpu.sync_copy(ones_buf, shist.at[idx_buf], add=True)
        return carry

    lax.fori_loop(0, nchunk, chunk_body, 0)
    plsc.subcore_barrier()

    @pl.when(sid == 0)
    def _():
        pltpu.sync_copy(shist, out_hbm.at[pl.ds(cid * hbins, hbins)])


def _reduce_kernel(inv_ref, hist_ref, out_ref, *, C):
    iu = jax.lax.broadcasted_iota(jnp.int32, (NLO, NLO), 0)
    ju = jax.lax.broadcasted_iota(jnp.int32, (NLO, NLO), 1)
    U = (iu <= ju).astype(jnp.float32)
    O = jnp.ones((NLO, NLO), jnp.float32)
    il = jax.lax.broadcasted_iota(jnp.int32, (NHI, NHI), 0)
    jl = jax.lax.broadcasted_iota(jnp.int32, (NHI, NHI), 1)
    L = (jl < il).astype(jnp.float32)

    def dn(a, bm):
        return jax.lax.dot_general(a, bm, (((1,), (0,)), ((), ())),
                                   preferred_element_type=jnp.float32)

    bidx = (jax.lax.broadcasted_iota(jnp.int32, (NHI, NLO), 0) * NLO +
            jax.lax.broadcasted_iota(jnp.int32, (NHI, NLO), 1))
    centers = ((1.0 - (bidx.astype(jnp.float32) + 0.5) / NBINS) /
               inv_ref[0, 0])

    total = jnp.float32(0.0)
    for c in range(C):
        h = hist_ref[0, c]
        for w in range(1, _NC):
            h = h + hist_ref[w, c]
        g = h[1]
        n = h[0] + g
        R = dn(n, U) + dn(L, dn(n, O))
        G = dn(g, U) + dn(L, dn(g, O))
        gts = jnp.sum(g)
        un_in = jnp.maximum(gts + R - G, 1.0)
        j_in = jnp.where(R > 0.5, 1.0 - (gts - G) / un_in, 0.0)
        Rx = R - n
        Gx = G - g
        un_ex = jnp.maximum(gts + Rx - Gx, 1.0)
        j_ex = jnp.where(Rx > 0.5, 1.0 - (gts - Gx) / un_ex, 0.0)
        total = total + jnp.sum(centers * (j_in - j_ex))
    out_ref[...] = jnp.broadcast_to(total / C, (1, 1))


def _lovasz_pallas(x, t):
    Bq, C, P = x.shape
    CH = _pick_chunk(P)
    nstep_inner = P // CH
    inv = (1.0 / (jnp.max(jnp.abs(x)) + 1.0)).reshape(1, 1)

    gidx = pl.pallas_call(
        functools.partial(_bin_kernel, C=C, CH=CH),
        grid=(Bq, nstep_inner),
        in_specs=[
            pl.BlockSpec(memory_space=pltpu.SMEM),
            pl.BlockSpec((1, C, CH), lambda b, i: (b, 0, i)),
            pl.BlockSpec((1, C, CH), lambda b, i: (b, 0, i)),
        ],
        out_specs=pl.BlockSpec((1, C, CH), lambda b, i: (b, 0, i)),
        out_shape=jax.ShapeDtypeStruct((Bq, C, P), jnp.int32),
    )(inv, x, t)

    items = Bq * C * P
    items_w = items // _NW
    hbins = C * 2 * NBINS
    chunk = 14592
    nchunk = items_w // chunk
    assert nchunk * chunk == items_w

    hist = pl.kernel(
        functools.partial(_sc_hist_kernel, hbins=hbins, items_w=items_w,
                          chunk=chunk, nchunk=nchunk),
        out_type=jax.ShapeDtypeStruct((_NC * hbins,), jnp.float32),
        mesh=plsc.VectorSubcoreMesh(core_axis_name="c",
                                    subcore_axis_name="s"),
        scratch_types=[
            pltpu.VMEM((chunk,), jnp.int32),
            pltpu.VMEM((chunk,), jnp.float32),
            pltpu.VMEM((hbins,), jnp.float32),
            pltpu.VMEM_SHARED((hbins,), jnp.float32),
        ],
    )(gidx.reshape(items))

    out = pl.pallas_call(
        functools.partial(_reduce_kernel, C=C),
        in_specs=[
            pl.BlockSpec(memory_space=pltpu.SMEM),
            pl.BlockSpec(memory_space=pltpu.VMEM),
        ],
        out_shape=jax.ShapeDtypeStruct((1, 1), jnp.float32),
    )(inv, hist.reshape(_NC, C, 2, NHI, NLO))
    return out[0, 0]


def kernel(pred, score, target):
    del score
    Bq, C = pred.shape[1], pred.shape[2]
    P = pred.shape[3] * pred.shape[4]
    x = pred.reshape(Bq, C, P).astype(jnp.float32)
    t = target.reshape(Bq, C, P).astype(jnp.float32)
    return _lovasz_pallas(x, t)


# speedup vs baseline: 1.1984x; 1.0010x over previous
import functools

import jax
import jax.numpy as jnp
from jax import lax
from jax.experimental import pallas as pl
from jax.experimental.pallas import tpu as pltpu
from jax.experimental.pallas import tpu_sc as plsc

NHI = 32
NLO = 64
NBINS = NHI * NLO

_NC = 2
_NS = 16
_NW = _NC * _NS


def _pick_chunk(P):
    for ch in (18432, 9216, 4608, 2304, 1152, 768, 512, 384, 256, 128):
        if P % ch == 0:
            return ch
    return P


def _bin_kernel(inv_ref, x_ref, t_ref, gidx_ref, *, C, CH):
    x = x_ref[0]
    t = t_ref[0]
    inv = inv_ref[0, 0]

    iota_c = jax.lax.broadcasted_iota(jnp.int32, (C, CH), 0)
    mxv = jnp.max(t, axis=0, keepdims=True)
    lab = jnp.min(jnp.where(t == mxv, iota_c, C), axis=0, keepdims=True)

    fg_all = (lab == iota_c)
    e_all = jnp.abs(fg_all.astype(jnp.float32) - x)
    idx = (NBINS * (1.0 - e_all * inv)).astype(jnp.int32)
    idx = jnp.clip(idx, 0, NBINS - 1)
    idx = jnp.where(fg_all, idx + NBINS, idx)
    gidx_ref[0] = idx + iota_c * (2 * NBINS)


def _sc_hist_kernel(gidx_hbm, out_hbm, idx_buf, ones_buf, tmp_buf, shist,
                    *, hbins, items_w, chunk, nchunk):
    cid = lax.axis_index("c")
    sid = lax.axis_index("s")
    wid = sid * _NC + cid
    base = wid * items_w
    zeros16 = jnp.zeros((16,), jnp.float32)
    ones16 = jnp.full((16,), 1.0, jnp.float32)

    def zbody(i, carry):
        tmp_buf[pl.ds(i * 16, 16)] = zeros16
        return carry

    lax.fori_loop(0, hbins // 16, zbody, 0)

    def obody(i, carry):
        ones_buf[pl.ds(i * 16, 16)] = ones16
        return carry

    lax.fori_loop(0, chunk // 16, obody, 0)

    @pl.when(sid == 0)
    def _():
        pltpu.sync_copy(tmp_buf, shist)

    plsc.subcore_barrier()

    def chunk_body(j, carry):
        pltpu.sync_copy(gidx_hbm.at[pl.ds(base + j * chunk, chunk)],
                        idx_buf)
        pltpu.sync_copy(ones_buf, shist.at[idx_buf], add=True)
        return carry

    lax.fori_loop(0, nchunk, chunk_body, 0)
    plsc.subcore_barrier()

    @pl.when(sid == 0)
    def _():
        pltpu.sync_copy(shist, out_hbm.at[pl.ds(cid * hbins, hbins)])


def _reduce_kernel(inv_ref, hist_ref, out_ref, *, C):
    iu = jax.lax.broadcasted_iota(jnp.int32, (NLO, NLO), 0)
    ju = jax.lax.broadcasted_iota(jnp.int32, (NLO, NLO), 1)
    U = (iu <= ju).astype(jnp.float32)
    O = jnp.ones((NLO, NLO), jnp.float32)
    il = jax.lax.broadcasted_iota(jnp.int32, (NHI, NHI), 0)
    jl = jax.lax.broadcasted_iota(jnp.int32, (NHI, NHI), 1)
    L = (jl < il).astype(jnp.float32)

    def dn(a, bm):
        return jax.lax.dot_general(a, bm, (((1,), (0,)), ((), ())),
                                   preferred_element_type=jnp.float32)

    bidx = (jax.lax.broadcasted_iota(jnp.int32, (NHI, NLO), 0) * NLO +
            jax.lax.broadcasted_iota(jnp.int32, (NHI, NLO), 1))
    centers = ((1.0 - (bidx.astype(jnp.float32) + 0.5) / NBINS) /
               inv_ref[0, 0])

    total = jnp.float32(0.0)
    for c in range(C):
        h = hist_ref[0, c]
        for w in range(1, _NC):
            h = h + hist_ref[w, c]
        g = h[1]
        n = h[0] + g
        R = dn(n, U) + dn(L, dn(n, O))
        G = dn(g, U) + dn(L, dn(g, O))
        gts = jnp.sum(g)
        un_in = jnp.maximum(gts + R - G, 1.0)
        j_in = jnp.where(R > 0.5, 1.0 - (gts - G) / un_in, 0.0)
        Rx = R - n
        Gx = G - g
        un_ex = jnp.maximum(gts + Rx - Gx, 1.0)
        j_ex = jnp.where(Rx > 0.5, 1.0 - (gts - Gx) / un_ex, 0.0)
        total = total + jnp.sum(centers * (j_in - j_ex))
    out_ref[...] = jnp.broadcast_to(total / C, (1, 1))


def _lovasz_pallas(x, t):
    Bq, C, P = x.shape
    CH = _pick_chunk(P)
    nstep_inner = P // CH
    inv = (1.0 / (jnp.max(jnp.abs(x)) + 1.0)).reshape(1, 1)

    gidx = pl.pallas_call(
        functools.partial(_bin_kernel, C=C, CH=CH),
        grid=(Bq, nstep_inner),
        in_specs=[
            pl.BlockSpec(memory_space=pltpu.SMEM),
            pl.BlockSpec((1, C, CH), lambda b, i: (b, 0, i)),
            pl.BlockSpec((1, C, CH), lambda b, i: (b, 0, i)),
        ],
        out_specs=pl.BlockSpec((1, C, CH), lambda b, i: (b, 0, i)),
        out_shape=jax.ShapeDtypeStruct((Bq, C, P), jnp.int32),
    )(inv, x, t)

    items = Bq * C * P
    items_w = items // _NW
    hbins = C * 2 * NBINS
    chunk = 14592
    nchunk = items_w // chunk
    assert nchunk * chunk == items_w

    hist = pl.kernel(
        functools.partial(_sc_hist_kernel, hbins=hbins, items_w=items_w,
                          chunk=chunk, nchunk=nchunk),
        out_type=jax.ShapeDtypeStruct((_NC * hbins,), jnp.float32),
        mesh=plsc.VectorSubcoreMesh(core_axis_name="c",
                                    subcore_axis_name="s"),
        scratch_types=[
            pltpu.VMEM((chunk,), jnp.int32),
            pltpu.VMEM((chunk,), jnp.float32),
            pltpu.VMEM((hbins,), jnp.float32),
            pltpu.VMEM_SHARED((hbins,), jnp.float32),
        ],
    )(gidx.reshape(items))

    out = pl.pallas_call(
        functools.partial(_reduce_kernel, C=C),
        in_specs=[
            pl.BlockSpec(memory_space=pltpu.SMEM),
            pl.BlockSpec(memory_space=pltpu.VMEM),
        ],
        out_shape=jax.ShapeDtypeStruct((1, 1), jnp.float32),
    )(inv, hist.reshape(_NC, C, 2, NHI, NLO))
    return out[0, 0]


def kernel(pred, score, target):
    del score
    Bq, C = pred.shape[1], pred.shape[2]
    P = pred.shape[3] * pred.shape[4]
    x = pred.reshape(Bq, C, P).astype(jnp.float32)
    t = target.reshape(Bq, C, P).astype(jnp.float32)
    return _lovasz_pallas(x, t)


# speedup vs baseline: 1.4339x; 1.1965x over previous
import functools

import jax
import jax.numpy as jnp
from jax import lax
from jax.experimental import pallas as pl
from jax.experimental.pallas import tpu as pltpu
from jax.experimental.pallas import tpu_sc as plsc

NHI = 32
NLO = 64
NBINS = NHI * NLO

_NC = 2
_NS = 16
_NW = _NC * _NS


def _pick_chunk(P):
    for ch in (18432, 9216, 4608, 2304, 1152, 768, 512, 384, 256, 128):
        if P % ch == 0:
            return ch
    return P


def _bin_kernel(inv_ref, x_ref, t_ref, gidx_ref, *, C, CH):
    x = x_ref[0]
    t = t_ref[0]
    inv = inv_ref[0, 0]

    iota_c = jax.lax.broadcasted_iota(jnp.int32, (C, CH), 0)
    mxv = jnp.max(t, axis=0, keepdims=True)
    lab = jnp.min(jnp.where(t == mxv, iota_c, C), axis=0, keepdims=True)

    fg_all = (lab == iota_c)
    e_all = jnp.abs(fg_all.astype(jnp.float32) - x)
    idx = (NBINS * (1.0 - e_all * inv)).astype(jnp.int32)
    idx = jnp.clip(idx, 0, NBINS - 1)
    idx = jnp.where(fg_all, idx + NBINS, idx)
    gidx_ref[0] = idx + iota_c * (2 * NBINS)


def _sc_hist_kernel(gidx_hbm, out_hbm, idx_buf, ones_buf, tmp_buf, shist,
                    *, hbins, items_w, chunk, nchunk):
    cid = lax.axis_index("c")
    sid = lax.axis_index("s")
    wid = sid * _NC + cid
    base = wid * items_w
    zeros16 = jnp.zeros((16,), jnp.float32)
    ones16 = jnp.full((16,), 1.0, jnp.float32)

    def zbody(i, carry):
        tmp_buf[pl.ds(i * 16, 16)] = zeros16
        return carry

    lax.fori_loop(0, hbins // 16, zbody, 0)

    def obody(i, carry):
        ones_buf[pl.ds(i * 16, 16)] = ones16
        return carry

    lax.fori_loop(0, chunk // 16, obody, 0)

    @pl.when(sid == 0)
    def _():
        pltpu.sync_copy(tmp_buf, shist)

    plsc.subcore_barrier()

    def chunk_body(j, carry):
        pltpu.sync_copy(gidx_hbm.at[pl.ds(base + j * chunk, chunk)],
                        idx_buf)
        pltpu.sync_copy(ones_buf, shist.at[idx_buf], add=True)
        return carry

    lax.fori_loop(0, nchunk, chunk_body, 0)
    plsc.subcore_barrier()

    @pl.when(sid == 0)
    def _():
        pltpu.sync_copy(shist, out_hbm.at[pl.ds(cid * hbins, hbins)])


def _tc_hist_kernel(inv_ref, x_ref, t_ref, hist_ref, *, C, CH):
    i = pl.program_id(1)

    @pl.when(i == 0)
    def _():
        hist_ref[...] = jnp.zeros_like(hist_ref)

    x = x_ref[0]
    t = t_ref[0]
    inv = inv_ref[0, 0]

    iota_c = jax.lax.broadcasted_iota(jnp.int32, (C, CH), 0)
    mxv = jnp.max(t, axis=0, keepdims=True)
    lab = jnp.min(jnp.where(t == mxv, iota_c, C), axis=0, keepdims=True)

    fg_all = (lab == iota_c)
    e_all = jnp.abs(fg_all.astype(jnp.float32) - x)
    idx = (NBINS * (1.0 - e_all * inv)).astype(jnp.int32)
    idx = jnp.clip(idx, 0, NBINS - 1)
    hi_all = jnp.right_shift(idx, 6)
    hi_all = jnp.where(fg_all, hi_all + NHI, hi_all)
    lo_all = jnp.bitwise_and(idx, NLO - 1)

    iota_hi = jax.lax.broadcasted_iota(jnp.int32, (2 * NHI, CH), 0)
    iota_lo = jax.lax.broadcasted_iota(jnp.int32, (NLO, CH), 0)
    for c in range(C):
        ohi = (hi_all[c:c + 1] == iota_hi).astype(jnp.bfloat16)
        olo = (lo_all[c:c + 1] == iota_lo).astype(jnp.bfloat16)
        upd = jax.lax.dot_general(
            ohi, olo, (((1,), (1,)), ((), ())),
            preferred_element_type=jnp.float32)
        hist_ref[0, c] = hist_ref[0, c] + upd


def _reduce_kernel(inv_ref, hsc_ref, htc_ref, out_ref, *, C, NB):
    iu = jax.lax.broadcasted_iota(jnp.int32, (NLO, NLO), 0)
    ju = jax.lax.broadcasted_iota(jnp.int32, (NLO, NLO), 1)
    U = (iu <= ju).astype(jnp.float32)
    O = jnp.ones((NLO, NLO), jnp.float32)
    il = jax.lax.broadcasted_iota(jnp.int32, (NHI, NHI), 0)
    jl = jax.lax.broadcasted_iota(jnp.int32, (NHI, NHI), 1)
    L = (jl < il).astype(jnp.float32)

    def dn(a, bm):
        return jax.lax.dot_general(a, bm, (((1,), (0,)), ((), ())),
                                   preferred_element_type=jnp.float32)

    bidx = (jax.lax.broadcasted_iota(jnp.int32, (NHI, NLO), 0) * NLO +
            jax.lax.broadcasted_iota(jnp.int32, (NHI, NLO), 1))
    centers = ((1.0 - (bidx.astype(jnp.float32) + 0.5) / NBINS) /
               inv_ref[0, 0])

    total = jnp.float32(0.0)
    for c in range(C):
        h = hsc_ref[0, c]
        for w in range(1, _NC):
            h = h + hsc_ref[w, c]
        for b in range(NB):
            h = h + htc_ref[b, c]
        g = h[1]
        n = h[0] + g
        R = dn(n, U) + dn(L, dn(n, O))
        G = dn(g, U) + dn(L, dn(g, O))
        gts = jnp.sum(g)
        un_in = jnp.maximum(gts + R - G, 1.0)
        j_in = jnp.where(R > 0.5, 1.0 - (gts - G) / un_in, 0.0)
        Rx = R - n
        Gx = G - g
        un_ex = jnp.maximum(gts + Rx - Gx, 1.0)
        j_ex = jnp.where(Rx > 0.5, 1.0 - (gts - Gx) / un_ex, 0.0)
        total = total + jnp.sum(centers * (j_in - j_ex))
    out_ref[...] = jnp.broadcast_to(total / C, (1, 1))


def _lovasz_pallas(x, t):
    Bq, C, P = x.shape
    CH = _pick_chunk(P)
    nstep = P // CH
    ns_sc = max(1, (5 * nstep) // 8)
    if Bq * C * (ns_sc * CH) % (_NW * 16):
        ns_sc = nstep
    ns_tc = nstep - ns_sc
    Ps = ns_sc * CH
    inv = (1.0 / (jnp.max(jnp.abs(x)) + 1.0)).reshape(1, 1)

    gidx = pl.pallas_call(
        functools.partial(_bin_kernel, C=C, CH=CH),
        grid=(Bq, ns_sc),
        in_specs=[
            pl.BlockSpec(memory_space=pltpu.SMEM),
            pl.BlockSpec((1, C, CH), lambda b, i: (b, 0, i)),
            pl.BlockSpec((1, C, CH), lambda b, i: (b, 0, i)),
        ],
        out_specs=pl.BlockSpec((1, C, CH), lambda b, i: (b, 0, i)),
        out_shape=jax.ShapeDtypeStruct((Bq, C, Ps), jnp.int32),
    )(inv, x, t)

    items = Bq * C * Ps
    items_w = items // _NW
    hbins = C * 2 * NBINS
    chunk = items_w
    for nch in range(2, 64):
        if items_w % nch == 0 and items_w // nch <= 16384 \
                and (items_w // nch) % 16 == 0:
            chunk = items_w // nch
            break
    nchunk = items_w // chunk

    hist_sc = pl.kernel(
        functools.partial(_sc_hist_kernel, hbins=hbins, items_w=items_w,
                          chunk=chunk, nchunk=nchunk),
        out_type=jax.ShapeDtypeStruct((_NC * hbins,), jnp.float32),
        mesh=plsc.VectorSubcoreMesh(core_axis_name="c",
                                    subcore_axis_name="s"),
        scratch_types=[
            pltpu.VMEM((chunk,), jnp.int32),
            pltpu.VMEM((chunk,), jnp.float32),
            pltpu.VMEM((hbins,), jnp.float32),
            pltpu.VMEM_SHARED((hbins,), jnp.float32),
        ],
    )(gidx.reshape(items))

    if ns_tc:
        hist_tc = pl.pallas_call(
            functools.partial(_tc_hist_kernel, C=C, CH=CH),
            grid=(Bq, ns_tc),
            in_specs=[
                pl.BlockSpec(memory_space=pltpu.SMEM),
                pl.BlockSpec((1, C, CH), lambda b, i: (b, 0, i + ns_sc)),
                pl.BlockSpec((1, C, CH), lambda b, i: (b, 0, i + ns_sc)),
            ],
            out_specs=pl.BlockSpec((1, C, 2 * NHI, NLO),
                                   lambda b, i: (b, 0, 0, 0)),
            out_shape=jax.ShapeDtypeStruct((Bq, C, 2 * NHI, NLO),
                                           jnp.float32),
        )(inv, x, t)
    else:
        hist_tc = jnp.zeros((Bq, C, 2 * NHI, NLO), jnp.float32)

    out = pl.pallas_call(
        functools.partial(_reduce_kernel, C=C, NB=Bq),
        in_specs=[
            pl.BlockSpec(memory_space=pltpu.SMEM),
            pl.BlockSpec(memory_space=pltpu.VMEM),
            pl.BlockSpec(memory_space=pltpu.VMEM),
        ],
        out_shape=jax.ShapeDtypeStruct((1, 1), jnp.float32),
    )(inv, hist_sc.reshape(_NC, C, 2, NHI, NLO),
      hist_tc.reshape(Bq, C, 2, NHI, NLO))
    return out[0, 0]


def kernel(pred, score, target):
    del score
    Bq, C = pred.shape[1], pred.shape[2]
    P = pred.shape[3] * pred.shape[4]
    x = pred.reshape(Bq, C, P).astype(jnp.float32)
    t = target.reshape(Bq, C, P).astype(jnp.float32)
    return _lovasz_pallas(x, t)
